# baseline (speedup 1.0000x reference)
import dataclasses
import functools

import jax
import jax.numpy as jnp
from jax import lax
from jax.experimental import pallas as pl
from jax.experimental.pallas import tpu as pltpu
from jax.experimental.pallas import tpu_sc as plsc

L = 16
U = 4
R = 2


def _gather_rows(B, C, N, M):
  info = plsc.get_sparse_core_info()
  NC, NS = info.num_cores, info.num_subcores
  NW = NC * NS
  ROWS = B * C
  assert ROWS % (NW * R) == 0
  CPW = ROWS // NW // R
  RPW = CPW * R
  assert (C % RPW == 0) or (RPW % C == 0)
  assert M % (L * U) == 0

  mesh = plsc.VectorSubcoreMesh(core_axis_name="c", subcore_axis_name="s")

  cp = pltpu.CompilerParams()
  if "needs_layout_passes" in pltpu.CompilerParams.__dataclass_fields__:
    cp = dataclasses.replace(cp, needs_layout_passes=False)

  @functools.partial(
      pl.kernel,
      compiler_params=cp,
      out_type=jax.ShapeDtypeStruct((ROWS // R, R * M), jnp.float32),
      mesh=mesh,
      scratch_types=[
          pltpu.VMEM((M,), jnp.int32),
          pltpu.VMEM((R * N,), jnp.float32),
          pltpu.VMEM((R * N,), jnp.float32),
          pltpu.VMEM((R * M,), jnp.float32),
          pltpu.VMEM((R * M,), jnp.float32),
          pltpu.SemaphoreType.DMA((2,)),
          pltpu.SemaphoreType.DMA((2,)),
      ],
  )
  def k(f_hbm, i_hbm, o_hbm, idx_v, row_a, row_b, out_a, out_b,
        sem_in, sem_out):
    wid = lax.axis_index("s") * NC + lax.axis_index("c")
    ck0 = wid * CPW
    b = (ck0 * R) // C
    rows = (row_a, row_b)
    outs = (out_a, out_b)

    pltpu.sync_copy(i_hbm.at[b], idx_v)

    pltpu.async_copy(f_hbm.at[ck0], row_a, sem_in.at[0])
    pltpu.async_copy(f_hbm.at[ck0 + 1], row_b, sem_in.at[1])

    @pl.loop(0, CPW, step=2)
    def _(g):
      for p in range(2):
        r = g + p
        pltpu.make_async_copy(f_hbm.at[ck0 + r], rows[p],
                              sem_in.at[p]).wait()

        @pl.when(r >= 2)
        def _():
          pltpu.make_async_copy(outs[p], o_hbm.at[ck0 + r - 2],
                                sem_out.at[p]).wait()

        @pl.loop(0, M, step=L * U)
        def _(i):
          for u in range(U):
            off = i + u * L
            idxv = idx_v[pl.ds(off, L)]
            outs[p][pl.ds(off, L)] = plsc.load_gather(rows[p], [idxv])
            outs[p][pl.ds(M + off, L)] = plsc.load_gather(
                rows[p], [idxv + N])

        pltpu.async_copy(outs[p], o_hbm.at[ck0 + r], sem_out.at[p])

        @pl.when(r + 2 < CPW)
        def _():
          pltpu.async_copy(f_hbm.at[ck0 + r + 2], rows[p], sem_in.at[p])

    pltpu.make_async_copy(out_a, o_hbm.at[ck0 + CPW - 2],
                          sem_out.at[0]).wait()
    pltpu.make_async_copy(out_b, o_hbm.at[ck0 + CPW - 1],
                          sem_out.at[1]).wait()

  return k


@jax.jit
def kernel(features, indices):
  B, C, N = features.shape
  M = indices.shape[1]
  k = _gather_rows(B, C, N, M)
  out = k(features.reshape(B * C // R, R * N), indices)
  return out.reshape(B, C, M)


# speedup vs baseline: 2.0128x; 2.0128x over previous
import dataclasses
import functools

import jax
import jax.numpy as jnp
from jax import lax
from jax.experimental import pallas as pl
from jax.experimental.pallas import tpu as pltpu
from jax.experimental.pallas import tpu_sc as plsc

L = 16
U = 4
R = 2


def _gather_rows(B, C, N, M):
  info = plsc.get_sparse_core_info()
  NC, NS = info.num_cores, info.num_subcores
  NW = NC * NS
  ROWS = B * C
  assert ROWS % (NW * R) == 0
  CPW = ROWS // NW // R
  RPW = CPW * R
  assert (C % RPW == 0) or (RPW % C == 0)
  assert M % (L * U) == 0

  mesh = plsc.VectorSubcoreMesh(core_axis_name="c", subcore_axis_name="s")

  cp = pltpu.CompilerParams()
  if "needs_layout_passes" in pltpu.CompilerParams.__dataclass_fields__:
    cp = dataclasses.replace(cp, needs_layout_passes=False)

  @functools.partial(
      pl.kernel,
      compiler_params=cp,
      out_type=jax.ShapeDtypeStruct((ROWS, M), jnp.float32),
      mesh=mesh,
      scratch_types=[
          pltpu.VMEM((M,), jnp.int32),
          pltpu.VMEM((R, N), jnp.float32),
          pltpu.VMEM((R, N), jnp.float32),
          pltpu.VMEM((R, M), jnp.float32),
          pltpu.VMEM((R, M), jnp.float32),
          pltpu.SemaphoreType.DMA((2,)),
          pltpu.SemaphoreType.DMA((2,)),
      ],
  )
  def k(f_hbm, i_hbm, o_hbm, idx_v, row_a, row_b, out_a, out_b,
        sem_in, sem_out):
    wid = lax.axis_index("s") * NC + lax.axis_index("c")
    row0 = wid * RPW
    b = row0 // C
    rows = (row_a, row_b)
    outs = (out_a, out_b)
    rsel = [jnp.full((L,), j, jnp.int32) for j in range(R)]

    pltpu.sync_copy(i_hbm.at[b], idx_v)

    pltpu.async_copy(f_hbm.at[pl.ds(row0, R)], row_a, sem_in.at[0])
    pltpu.async_copy(f_hbm.at[pl.ds(row0 + R, R)], row_b, sem_in.at[1])

    @pl.loop(0, CPW, step=2)
    def _(g):
      for p in range(2):
        r = g + p
        pltpu.make_async_copy(f_hbm.at[pl.ds(row0 + r * R, R)], rows[p],
                              sem_in.at[p]).wait()

        @pl.when(r >= 2)
        def _():
          pltpu.make_async_copy(outs[p], o_hbm.at[pl.ds(
              row0 + (r - 2) * R, R)], sem_out.at[p]).wait()

        @pl.loop(0, M, step=L * U)
        def _(i):
          for u in range(U):
            off = i + u * L
            idxv = idx_v[pl.ds(off, L)]
            for j in range(R):
              outs[p][j, pl.ds(off, L)] = plsc.load_gather(
                  rows[p], [rsel[j], idxv])

        pltpu.async_copy(outs[p], o_hbm.at[pl.ds(row0 + r * R, R)],
                         sem_out.at[p])

        @pl.when(r + 2 < CPW)
        def _():
          pltpu.async_copy(f_hbm.at[pl.ds(row0 + (r + 2) * R, R)],
                           rows[p], sem_in.at[p])

    pltpu.make_async_copy(out_a, o_hbm.at[pl.ds(row0 + (CPW - 2) * R, R)],
                          sem_out.at[0]).wait()
    pltpu.make_async_copy(out_b, o_hbm.at[pl.ds(row0 + (CPW - 1) * R, R)],
                          sem_out.at[1]).wait()

  return k


@jax.jit
def kernel(features, indices):
  B, C, N = features.shape
  M = indices.shape[1]
  k = _gather_rows(B, C, N, M)
  out = k(features.reshape(B * C, N), indices)
  return out.reshape(B, C, M)


# speedup vs baseline: 2.3015x; 1.1434x over previous
import dataclasses
import functools

import jax
import jax.numpy as jnp
from jax import lax
from jax.experimental import pallas as pl
from jax.experimental.pallas import tpu as pltpu
from jax.experimental.pallas import tpu_sc as plsc

L = 16
U = 8
R = 2


def _gather_rows(B, C, N, M):
  info = plsc.get_sparse_core_info()
  NC, NS = info.num_cores, info.num_subcores
  NW = NC * NS
  ROWS = B * C
  assert ROWS % (NW * R) == 0
  CPW = ROWS // NW // R
  RPW = CPW * R
  assert (C % RPW == 0) or (RPW % C == 0)
  assert M % (L * U) == 0

  mesh = plsc.VectorSubcoreMesh(core_axis_name="c", subcore_axis_name="s")

  cp = pltpu.CompilerParams()
  if "needs_layout_passes" in pltpu.CompilerParams.__dataclass_fields__:
    cp = dataclasses.replace(cp, needs_layout_passes=False)

  @functools.partial(
      pl.kernel,
      compiler_params=cp,
      out_type=jax.ShapeDtypeStruct((ROWS, M), jnp.float32),
      mesh=mesh,
      scratch_types=[
          pltpu.VMEM((M,), jnp.int32),
          pltpu.VMEM((R * N,), jnp.float32),
          pltpu.VMEM((R * N,), jnp.float32),
          pltpu.VMEM((R, M), jnp.float32),
          pltpu.VMEM((R, M), jnp.float32),
          pltpu.SemaphoreType.DMA((2, R)),
          pltpu.SemaphoreType.DMA((2,)),
      ],
  )
  def k(f_hbm, i_hbm, o_hbm, idx_v, row_a, row_b, out_a, out_b,
        sem_in, sem_out):
    wid = lax.axis_index("s") * NC + lax.axis_index("c")
    row0 = wid * RPW
    b = row0 // C
    rows = (row_a, row_b)
    outs = (out_a, out_b)

    pltpu.sync_copy(i_hbm.at[b], idx_v)

    def start_in(r, p):
      for j in range(R):
        pltpu.async_copy(f_hbm.at[row0 + r * R + j],
                         rows[p].at[pl.ds(j * N, N)], sem_in.at[p, j])

    def wait_in(r, p):
      for j in range(R):
        pltpu.make_async_copy(f_hbm.at[row0 + r * R + j],
                              rows[p].at[pl.ds(j * N, N)],
                              sem_in.at[p, j]).wait()

    start_in(0, 0)
    start_in(1, 1)

    @pl.loop(0, CPW, step=2)
    def _(g):
      for p in range(2):
        r = g + p
        wait_in(r, p)

        @pl.when(r >= 2)
        def _():
          pltpu.make_async_copy(outs[p], o_hbm.at[pl.ds(
              row0 + (r - 2) * R, R)], sem_out.at[p]).wait()

        @pl.loop(0, M, step=L * U)
        def _(i):
          for u in range(U):
            off = i + u * L
            idxv = idx_v[pl.ds(off, L)]
            for j in range(R):
              outs[p][j, pl.ds(off, L)] = plsc.load_gather(
                  rows[p], [idxv + (j * N)])

        pltpu.async_copy(outs[p], o_hbm.at[pl.ds(row0 + r * R, R)],
                         sem_out.at[p])

        @pl.when(r + 2 < CPW)
        def _():
          start_in(r + 2, p)

    pltpu.make_async_copy(out_a, o_hbm.at[pl.ds(row0 + (CPW - 2) * R, R)],
                          sem_out.at[0]).wait()
    pltpu.make_async_copy(out_b, o_hbm.at[pl.ds(row0 + (CPW - 1) * R, R)],
                          sem_out.at[1]).wait()

  return k


@jax.jit
def kernel(features, indices):
  B, C, N = features.shape
  M = indices.shape[1]
  k = _gather_rows(B, C, N, M)
  out = k(features.reshape(B * C, N), indices)
  return out.reshape(B, C, M)


# speedup vs baseline: 2.8922x; 1.2567x over previous
import dataclasses
import functools

import jax
import jax.numpy as jnp
from jax import lax
from jax.experimental import pallas as pl
from jax.experimental.pallas import tpu as pltpu
from jax.experimental.pallas import tpu_sc as plsc

UNROLL = 8


def _gather_rows(B, C, N, M):
  info = plsc.get_sparse_core_info()
  NC, NS = info.num_cores, info.num_subcores
  NW = NC * NS
  ROWS = B * C
  assert ROWS % NW == 0
  RPW = ROWS // NW
  assert (C % RPW == 0) or (RPW % C == 0)
  assert RPW % 2 == 0 and M % (16 * UNROLL) == 0

  mesh = plsc.VectorSubcoreMesh(core_axis_name="c", subcore_axis_name="s")

  cp = pltpu.CompilerParams()
  if "needs_layout_passes" in pltpu.CompilerParams.__dataclass_fields__:
    cp = dataclasses.replace(cp, needs_layout_passes=False)

  @functools.partial(
      pl.kernel,
      compiler_params=cp,
      out_type=jax.ShapeDtypeStruct((ROWS, M), jnp.float32),
      mesh=mesh,
      scratch_types=[
          pltpu.VMEM((M,), jnp.int32),
          pltpu.VMEM((N,), jnp.float32),
          pltpu.VMEM((N,), jnp.float32),
          pltpu.VMEM((M,), jnp.float32),
          pltpu.VMEM((M,), jnp.float32),
          pltpu.SemaphoreType.DMA((2,)),
          pltpu.SemaphoreType.DMA((2,)),
      ],
  )
  def k(f_hbm, i_hbm, o_hbm, idx_v, row0, row1, out0, out1, sem_r, sem_o):
    wid = lax.axis_index("s") * NC + lax.axis_index("c")
    r0 = wid * RPW
    b = r0 // C

    pltpu.sync_copy(i_hbm.at[b], idx_v)

    rows = [row0, row1]
    outs = [out0, out1]

    for p in range(2):
      pltpu.async_copy(f_hbm.at[r0 + p], rows[p], sem_r.at[p])

    @pl.loop(0, RPW, step=2)
    def _(g):
      for p in range(2):
        r = g + p
        pltpu.make_async_copy(f_hbm.at[r0 + r], rows[p], sem_r.at[p]).wait()

        @pl.when(r >= 2)
        def _():
          pltpu.make_async_copy(outs[p], o_hbm.at[r0 + r - 2],
                                sem_o.at[p]).wait()

        @pl.loop(0, M, step=16 * UNROLL)
        def _(j):
          for u in range(UNROLL):
            s = pl.ds(j + u * 16, 16)
            outs[p][s] = plsc.load_gather(rows[p], [idx_v[s]])

        pltpu.async_copy(outs[p], o_hbm.at[r0 + r], sem_o.at[p])

        @pl.when(r + 2 < RPW)
        def _():
          pltpu.async_copy(f_hbm.at[r0 + r + 2], rows[p], sem_r.at[p])

    for p in range(2):
      pltpu.make_async_copy(outs[p], o_hbm.at[r0 + RPW - 2 + p],
                            sem_o.at[p]).wait()

  return k


@jax.jit
def kernel(features, indices):
  B, C, N = features.shape
  M = indices.shape[1]
  k = _gather_rows(B, C, N, M)
  out = k(features.reshape(B * C, N), indices)
  return out.reshape(B, C, M)


# speedup vs baseline: 3.3573x; 1.1608x over previous
import dataclasses
import functools

import jax
import jax.numpy as jnp
from jax import lax
from jax.experimental import pallas as pl
from jax.experimental.pallas import tpu as pltpu
from jax.experimental.pallas import tpu_sc as plsc

UNROLL = 8


def _gather_rows(B, C, N, M):
  info = plsc.get_sparse_core_info()
  NC, NS = info.num_cores, info.num_subcores
  NW = NC * NS
  ROWS = B * C
  assert ROWS % NW == 0
  RPW = ROWS // NW
  assert (C % RPW == 0) or (RPW % C == 0)
  assert RPW % 2 == 0 and M % (16 * UNROLL) == 0

  mesh = plsc.VectorSubcoreMesh(core_axis_name="c", subcore_axis_name="s")

  cp = pltpu.CompilerParams()
  if "needs_layout_passes" in pltpu.CompilerParams.__dataclass_fields__:
    cp = dataclasses.replace(cp, needs_layout_passes=False)

  @functools.partial(
      pl.kernel,
      compiler_params=cp,
      out_type=jax.ShapeDtypeStruct((ROWS, M), jnp.float32),
      mesh=mesh,
      scratch_types=[
          pltpu.VMEM((M,), jnp.int32),
          pltpu.VMEM((N,), jnp.float32),
          pltpu.VMEM((N,), jnp.float32),
          pltpu.VMEM((M,), jnp.float32),
          pltpu.VMEM((M,), jnp.float32),
          pltpu.SemaphoreType.DMA((2,)),
          pltpu.SemaphoreType.DMA((2,)),
      ],
  )
  def k(f_hbm, i_hbm, o_hbm, idx_v, row0, row1, out0, out1, sem_r, sem_o):
    wid = lax.axis_index("s") * NC + lax.axis_index("c")
    r0 = wid * RPW
    b = r0 // C

    pltpu.sync_copy(i_hbm.at[b], idx_v)

    rows = [row0, row1]
    outs = [out0, out1]

    for p in range(2):
      pltpu.async_copy(f_hbm.at[r0 + p], rows[p], sem_r.at[p])

    @pl.loop(0, RPW, step=2)
    def _(g):
      for p in range(2):
        r = g + p
        pltpu.make_async_copy(f_hbm.at[r0 + r], rows[p], sem_r.at[p]).wait()

        @pl.when(r >= 2)
        def _():
          pltpu.make_async_copy(outs[p], o_hbm.at[r0 + r - 2],
                                sem_o.at[p]).wait()

        @plsc.parallel_loop(0, M, step=16, unroll=UNROLL)
        def _(j):
          s = pl.ds(j, 16)
          outs[p][s] = plsc.load_gather(rows[p], [idx_v[s]])

        pltpu.async_copy(outs[p], o_hbm.at[r0 + r], sem_o.at[p])

        @pl.when(r + 2 < RPW)
        def _():
          pltpu.async_copy(f_hbm.at[r0 + r + 2], rows[p], sem_r.at[p])

    for p in range(2):
      pltpu.make_async_copy(outs[p], o_hbm.at[r0 + RPW - 2 + p],
                            sem_o.at[p]).wait()

  return k


@jax.jit
def kernel(features, indices):
  B, C, N = features.shape
  M = indices.shape[1]
  k = _gather_rows(B, C, N, M)
  out = k(features.reshape(B * C, N), indices)
  return out.reshape(B, C, M)


# speedup vs baseline: 3.3605x; 1.0010x over previous
import dataclasses
import functools

import jax
import jax.numpy as jnp
from jax import lax
from jax.experimental import pallas as pl
from jax.experimental.pallas import tpu as pltpu
from jax.experimental.pallas import tpu_sc as plsc

UNROLL = 16


def _gather_rows(B, C, N, M):
  info = plsc.get_sparse_core_info()
  NC, NS = info.num_cores, info.num_subcores
  NW = NC * NS
  ROWS = B * C
  assert ROWS % NW == 0
  RPW = ROWS // NW
  assert (C % RPW == 0) or (RPW % C == 0)
  assert RPW % 2 == 0 and M % (16 * UNROLL) == 0

  mesh = plsc.VectorSubcoreMesh(core_axis_name="c", subcore_axis_name="s")

  cp = pltpu.CompilerParams()
  if "needs_layout_passes" in pltpu.CompilerParams.__dataclass_fields__:
    cp = dataclasses.replace(cp, needs_layout_passes=False)

  @functools.partial(
      pl.kernel,
      compiler_params=cp,
      out_type=jax.ShapeDtypeStruct((ROWS, M), jnp.float32),
      mesh=mesh,
      scratch_types=[
          pltpu.VMEM((M,), jnp.int32),
          pltpu.VMEM((N,), jnp.float32),
          pltpu.VMEM((N,), jnp.float32),
          pltpu.VMEM((M,), jnp.float32),
          pltpu.VMEM((M,), jnp.float32),
          pltpu.SemaphoreType.DMA((2,)),
          pltpu.SemaphoreType.DMA((2,)),
      ],
  )
  def k(f_hbm, i_hbm, o_hbm, idx_v, row0, row1, out0, out1, sem_r, sem_o):
    wid = lax.axis_index("s") * NC + lax.axis_index("c")
    r0 = wid * RPW
    b = r0 // C

    pltpu.sync_copy(i_hbm.at[b], idx_v)

    rows = [row0, row1]
    outs = [out0, out1]

    for p in range(2):
      pltpu.async_copy(f_hbm.at[r0 + p], rows[p], sem_r.at[p])

    @pl.loop(0, RPW, step=2)
    def _(g):
      for p in range(2):
        r = g + p
        pltpu.make_async_copy(f_hbm.at[r0 + r], rows[p], sem_r.at[p]).wait()

        @pl.when(r >= 2)
        def _():
          pltpu.make_async_copy(outs[p], o_hbm.at[r0 + r - 2],
                                sem_o.at[p]).wait()

        @plsc.parallel_loop(0, M, step=16, unroll=UNROLL)
        def _(j):
          s = pl.ds(j, 16)
          outs[p][s] = plsc.load_gather(rows[p], [idx_v[s]])

        pltpu.async_copy(outs[p], o_hbm.at[r0 + r], sem_o.at[p])

        @pl.when(r + 2 < RPW)
        def _():
          pltpu.async_copy(f_hbm.at[r0 + r + 2], rows[p], sem_r.at[p])

    for p in range(2):
      pltpu.make_async_copy(outs[p], o_hbm.at[r0 + RPW - 2 + p],
                            sem_o.at[p]).wait()

  return k


@jax.jit
def kernel(features, indices):
  B, C, N = features.shape
  M = indices.shape[1]
  k = _gather_rows(B, C, N, M)
  out = k(features.reshape(B * C, N), indices)
  return out.reshape(B, C, M)


# speedup vs baseline: 3.8944x; 1.1589x over previous
import dataclasses
import functools

import jax
import jax.numpy as jnp
from jax import lax
from jax.experimental import pallas as pl
from jax.experimental.pallas import tpu as pltpu
from jax.experimental.pallas import tpu_sc as plsc

UNROLL = 16
NBUF = 4


def _gather_rows(B, C, N, M):
  info = plsc.get_sparse_core_info()
  NC, NS = info.num_cores, info.num_subcores
  NW = NC * NS
  ROWS = B * C
  assert ROWS % NW == 0
  RPW = ROWS // NW
  assert (C % RPW == 0) or (RPW % C == 0)
  assert RPW % NBUF == 0 and M % (16 * UNROLL) == 0

  mesh = plsc.VectorSubcoreMesh(core_axis_name="c", subcore_axis_name="s")

  cp = pltpu.CompilerParams()
  if "needs_layout_passes" in pltpu.CompilerParams.__dataclass_fields__:
    cp = dataclasses.replace(cp, needs_layout_passes=False)

  @functools.partial(
      pl.kernel,
      compiler_params=cp,
      out_type=jax.ShapeDtypeStruct((ROWS, M), jnp.float32),
      mesh=mesh,
      scratch_types=(
          [pltpu.VMEM((M,), jnp.int32)]
          + [pltpu.VMEM((N,), jnp.float32)] * NBUF
          + [pltpu.VMEM((M,), jnp.float32)] * NBUF
          + [pltpu.SemaphoreType.DMA((NBUF,)),
             pltpu.SemaphoreType.DMA((NBUF,))]
      ),
  )
  def k(f_hbm, i_hbm, o_hbm, idx_v, *bufs):
    rows = list(bufs[:NBUF])
    outs = list(bufs[NBUF:2 * NBUF])
    sem_r, sem_o = bufs[2 * NBUF], bufs[2 * NBUF + 1]

    wid = lax.axis_index("s") * NC + lax.axis_index("c")
    r0 = wid * RPW
    b = r0 // C

    pltpu.sync_copy(i_hbm.at[b], idx_v)

    for p in range(NBUF):
      pltpu.async_copy(f_hbm.at[r0 + p], rows[p], sem_r.at[p])

    @pl.loop(0, RPW, step=NBUF)
    def _(g):
      for p in range(NBUF):
        r = g + p
        pltpu.make_async_copy(f_hbm.at[r0 + r], rows[p], sem_r.at[p]).wait()

        @pl.when(r >= NBUF)
        def _():
          pltpu.make_async_copy(outs[p], o_hbm.at[r0 + r - NBUF],
                                sem_o.at[p]).wait()

        @plsc.parallel_loop(0, M, step=16, unroll=UNROLL)
        def _(j):
          s = pl.ds(j, 16)
          outs[p][s] = plsc.load_gather(rows[p], [idx_v[s]])

        pltpu.async_copy(outs[p], o_hbm.at[r0 + r], sem_o.at[p])

        @pl.when(r + NBUF < RPW)
        def _():
          pltpu.async_copy(f_hbm.at[r0 + r + NBUF], rows[p], sem_r.at[p])

    for p in range(NBUF):
      pltpu.make_async_copy(outs[p], o_hbm.at[r0 + RPW - NBUF + p],
                            sem_o.at[p]).wait()

  return k


@jax.jit
def kernel(features, indices):
  B, C, N = features.shape
  M = indices.shape[1]
  k = _gather_rows(B, C, N, M)
  out = k(features.reshape(B * C, N), indices)
  return out.reshape(B, C, M)
